# baseline (speedup 1.0000x reference)
import functools

import jax
import jax.numpy as jnp
from jax import lax
from jax.experimental import pallas as pl
from jax.experimental.pallas import tpu as pltpu
from jax.experimental.pallas import tpu_sc as plsc

N = 50000
K = 64
DE = 16
DN = 128

NC = 2
NS = 16
NW = NC * NS
CPT = K // NW
CHUNK = 2000
NCHUNK = N // CHUNK

NB_A = 512
NB_C = 2048


def _stage_a_body(eh_ref, mask_ref, x_ref, idx_ref, wmat_ref, bu_ref,
                  wT_ref, xT_ref, idxT_ref):
    dX = jnp.dot(eh_ref[...], wmat_ref[...],
                 preferred_element_type=jnp.float32)
    z = dX + bu_ref[0, 0]
    w = (1.0 / (1.0 + jnp.exp(-z))) * mask_ref[...]
    wn = w / (jnp.sum(w, axis=-1, keepdims=True) + 1e-6)
    wT_ref[...] = wn.T
    xT_ref[...] = x_ref[...].T
    idxT_ref[...] = idx_ref[...].T


def _stage_a(eh2, mask2, x2, idx2, wmat, bu):
    grid = (pl.cdiv(N, NB_A),)
    return pl.pallas_call(
        _stage_a_body,
        grid=grid,
        in_specs=[
            pl.BlockSpec((NB_A, K * DE), lambda i: (i, 0)),
            pl.BlockSpec((NB_A, K), lambda i: (i, 0)),
            pl.BlockSpec((NB_A, K), lambda i: (i, 0)),
            pl.BlockSpec((NB_A, K), lambda i: (i, 0)),
            pl.BlockSpec((K * DE, K), lambda i: (0, 0)),
            pl.BlockSpec(memory_space=pltpu.SMEM),
        ],
        out_specs=[
            pl.BlockSpec((K, NB_A), lambda i: (0, i)),
            pl.BlockSpec((K, NB_A), lambda i: (0, i)),
            pl.BlockSpec((K, NB_A), lambda i: (0, i)),
        ],
        out_shape=[
            jax.ShapeDtypeStruct((K, N), jnp.float32),
            jax.ShapeDtypeStruct((K, N), jnp.float32),
            jax.ShapeDtypeStruct((K, N), jnp.int32),
        ],
    )(eh2, mask2, x2, idx2, wmat, bu)


def _sc_body(wT, xT, idxT, out, xcols, idxb, wb, part):
    cid = lax.axis_index("c")
    sid = lax.axis_index("s")
    wid = sid * NC + cid
    r0 = pl.multiple_of(wid * CPT * N, 8)
    r1 = pl.multiple_of(r0 + N, 8)
    pltpu.sync_copy(xT.at[pl.ds(r0, N)], xcols.at[pl.ds(0, N)])
    pltpu.sync_copy(xT.at[pl.ds(r1, N)], xcols.at[pl.ds(N, N)])

    def chunk_body(j, carry):
        base = j * CHUNK
        pltpu.sync_copy(idxT.at[pl.ds(pl.multiple_of(r0 + base, 8), CHUNK)],
                        idxb.at[0])
        pltpu.sync_copy(idxT.at[pl.ds(pl.multiple_of(r1 + base, 8), CHUNK)],
                        idxb.at[1])
        pltpu.sync_copy(wT.at[pl.ds(pl.multiple_of(r0 + base, 8), CHUNK)],
                        wb.at[0])
        pltpu.sync_copy(wT.at[pl.ds(pl.multiple_of(r1 + base, 8), CHUNK)],
                        wb.at[1])

        def vec_body(i, c):
            s = pl.ds(i * 16, 16)
            i0 = idxb[0, s]
            i1 = idxb[1, s] + N
            x0 = plsc.load_gather(xcols, [i0])
            x1 = plsc.load_gather(xcols, [i1])
            part[s] = wb[0, s] * x0 + wb[1, s] * x1
            return c

        lax.fori_loop(0, CHUNK // 16, vec_body, 0)
        pltpu.sync_copy(
            part, out.at[pl.ds(pl.multiple_of(wid * N + base, 8), CHUNK)])
        return carry

    lax.fori_loop(0, NCHUNK, chunk_body, 0)


@functools.cache
def _sc_gather():
    return pl.kernel(
        _sc_body,
        out_type=jax.ShapeDtypeStruct((NW * N,), jnp.float32),
        mesh=plsc.VectorSubcoreMesh(core_axis_name="c", subcore_axis_name="s",
                                    num_cores=NC, num_subcores=NS),
        scratch_types=[
            pltpu.VMEM((CPT * N,), jnp.float32),
            pltpu.VMEM((CPT, CHUNK), jnp.int32),
            pltpu.VMEM((CPT, CHUNK), jnp.float32),
            pltpu.VMEM((CHUNK,), jnp.float32),
        ],
        compiler_params=pltpu.CompilerParams(use_tc_tiling_on_sc=False,
                                             needs_layout_passes=False),
    )


def _stage_c_body(part_ref, nh_ref, wr_ref, br_ref, out_ref):
    ps = jnp.sum(part_ref[...], axis=0)
    refine = jnp.sum(nh_ref[...] * wr_ref[...], axis=-1)
    out_ref[...] = (ps + refine + br_ref[0, 0])[None, :]


def _stage_c(partials, nh2, wr, br):
    grid = (pl.cdiv(N, NB_C),)
    return pl.pallas_call(
        _stage_c_body,
        grid=grid,
        in_specs=[
            pl.BlockSpec((NW, NB_C), lambda i: (0, i)),
            pl.BlockSpec((NB_C, DN), lambda i: (i, 0)),
            pl.BlockSpec((1, DN), lambda i: (0, 0)),
            pl.BlockSpec(memory_space=pltpu.SMEM),
        ],
        out_specs=pl.BlockSpec((1, NB_C), lambda i: (0, i)),
        out_shape=jax.ShapeDtypeStruct((1, N), jnp.float32),
    )(partials, nh2, wr, br)


def kernel(X, node_h, edge_h, edge_idx, mask_i, mask_ij, Wu_w, Wu_b, Wr_w, Wr_b):
    eh2 = edge_h.reshape(N, K * DE)
    mask2 = mask_ij.reshape(N, K)
    x2 = X.reshape(N, K)
    idx2 = edge_idx.reshape(N, K).astype(jnp.int32)
    nh2 = node_h.reshape(N, DN)
    wmat = jnp.kron(jnp.eye(K, dtype=jnp.float32), Wu_w.reshape(DE, 1))
    bu = Wu_b.reshape(1, 1)
    br = Wr_b.reshape(1, 1)

    wT, xT, idxT = _stage_a(eh2, mask2, x2, idx2, wmat, bu)
    partials = _sc_gather()(wT.reshape(-1), xT.reshape(-1), idxT.reshape(-1))
    out2 = _stage_c(partials.reshape(NW, N), nh2, Wr_w.reshape(1, DN), br)
    return out2.reshape(1, N, 1)


# speedup vs baseline: 1.3669x; 1.3669x over previous
import functools

import jax
import jax.numpy as jnp
from jax import lax
from jax.experimental import pallas as pl
from jax.experimental.pallas import tpu as pltpu
from jax.experimental.pallas import tpu_sc as plsc

N = 50000
K = 64
DE = 16
DN = 128

MP = 400
NP = MP * 128

NC = 2
NS = 16
NW = NC * NS
CPT = K // NW
CROWS = 20
NCHUNK = MP // CROWS

NB_A = 1024
MB_A = NB_A // 128
NB_C = 2048
MB_C = NB_C // 128


def _stage_a_body(eh_ref, mask_ref, x_ref, idx_ref, wmat_ref, bu_ref,
                  wT_ref, xT_ref, idxT_ref):
    dX = jnp.dot(eh_ref[...], wmat_ref[...],
                 preferred_element_type=jnp.float32)
    z = dX + bu_ref[0, 0]
    w = (1.0 / (1.0 + jnp.exp(-z))) * mask_ref[...]
    wn = w / (jnp.sum(w, axis=-1, keepdims=True) + 1e-6)
    wT_ref[...] = wn.T.reshape(K, MB_A, 128)
    xT_ref[...] = x_ref[...].T.reshape(K, MB_A, 128)
    idxT_ref[...] = idx_ref[...].T.reshape(K, MB_A, 128)


def _stage_a(eh2, mask2, x2, idx2, wmat, bu):
    grid = (pl.cdiv(N, NB_A),)
    return pl.pallas_call(
        _stage_a_body,
        grid=grid,
        in_specs=[
            pl.BlockSpec((NB_A, K * DE), lambda i: (i, 0)),
            pl.BlockSpec((NB_A, K), lambda i: (i, 0)),
            pl.BlockSpec((NB_A, K), lambda i: (i, 0)),
            pl.BlockSpec((NB_A, K), lambda i: (i, 0)),
            pl.BlockSpec((K * DE, K), lambda i: (0, 0)),
            pl.BlockSpec(memory_space=pltpu.SMEM),
        ],
        out_specs=[
            pl.BlockSpec((K, MB_A, 128), lambda i: (0, i, 0)),
            pl.BlockSpec((K, MB_A, 128), lambda i: (0, i, 0)),
            pl.BlockSpec((K, MB_A, 128), lambda i: (0, i, 0)),
        ],
        out_shape=[
            jax.ShapeDtypeStruct((K, MP, 128), jnp.float32),
            jax.ShapeDtypeStruct((K, MP, 128), jnp.float32),
            jax.ShapeDtypeStruct((K, MP, 128), jnp.int32),
        ],
    )(eh2, mask2, x2, idx2, wmat, bu)


def _sc_body(wT, xT, idxT, out, xcols, idxb, wb, part,
             isem0, isem1, osem0, osem1):
    cid = lax.axis_index("c")
    sid = lax.axis_index("s")
    wid = sid * NC + cid
    k0 = wid * CPT
    isems = (isem0, isem1)
    osems = (osem0, osem1)

    pltpu.sync_copy(xT.at[k0], xcols.at[pl.ds(0, MP)])
    pltpu.sync_copy(xT.at[k0 + 1], xcols.at[pl.ds(MP, MP)])

    def issue_loads(j, b):
        r0 = j * CROWS
        for c in range(CPT):
            pltpu.async_copy(idxT.at[k0 + c, pl.ds(r0, CROWS)],
                             idxb.at[b, c], isems[b])
            pltpu.async_copy(wT.at[k0 + c, pl.ds(r0, CROWS)],
                             wb.at[b, c], isems[b])

    def wait_loads(b):
        for c in range(CPT):
            pltpu.make_async_copy(idxT.at[k0 + c, pl.ds(0, CROWS)],
                                  idxb.at[b, c], isems[b]).wait()
            pltpu.make_async_copy(wT.at[k0 + c, pl.ds(0, CROWS)],
                                  wb.at[b, c], isems[b]).wait()

    def out_window(j):
        return out.at[wid, pl.ds(j * CROWS, CROWS)]

    issue_loads(0, 0)

    def super_body(jj, carry):
        for b in range(2):
            j = 2 * jj + b

            @pl.when(j + 1 < NCHUNK)
            def _():
                issue_loads(j + 1, 1 - b)

            @pl.when(j >= 2)
            def _():
                pltpu.make_async_copy(part.at[b], out_window(j - 2),
                                      osems[b]).wait()

            wait_loads(b)

            def vec_body(i, c2):
                r = i >> 3
                l = (i & 7) * 16
                s = pl.ds(l, 16)
                g0 = jnp.clip(idxb[b, 0, r, s], 0, N - 1)
                g1 = jnp.clip(idxb[b, 1, r, s], 0, N - 1)
                row0 = jax.lax.shift_right_logical(g0, 7)
                row1 = jax.lax.shift_right_logical(g1, 7) + MP
                lane0 = jax.lax.bitwise_and(g0, 127)
                lane1 = jax.lax.bitwise_and(g1, 127)
                x0 = plsc.load_gather(xcols, [row0, lane0])
                x1 = plsc.load_gather(xcols, [row1, lane1])
                part[b, r, s] = wb[b, 0, r, s] * x0 + wb[b, 1, r, s] * x1
                return c2

            lax.fori_loop(0, CROWS * 8, vec_body, 0)
            pltpu.async_copy(part.at[b], out_window(j), osems[b])
        return carry

    lax.fori_loop(0, NCHUNK // 2, super_body, 0)
    pltpu.make_async_copy(part.at[0], out_window(NCHUNK - 2), osems[0]).wait()
    pltpu.make_async_copy(part.at[1], out_window(NCHUNK - 1), osems[1]).wait()


@functools.cache
def _sc_gather():
    return pl.kernel(
        _sc_body,
        out_type=jax.ShapeDtypeStruct((NW, MP, 128), jnp.float32),
        mesh=plsc.VectorSubcoreMesh(core_axis_name="c", subcore_axis_name="s",
                                    num_cores=NC, num_subcores=NS),
        scratch_types=[
            pltpu.VMEM((CPT * MP, 128), jnp.float32),
            pltpu.VMEM((2, CPT, CROWS, 128), jnp.int32),
            pltpu.VMEM((2, CPT, CROWS, 128), jnp.float32),
            pltpu.VMEM((2, CROWS, 128), jnp.float32),
            pltpu.SemaphoreType.DMA,
            pltpu.SemaphoreType.DMA,
            pltpu.SemaphoreType.DMA,
            pltpu.SemaphoreType.DMA,
        ],
        compiler_params=pltpu.CompilerParams(use_tc_tiling_on_sc=False,
                                             needs_layout_passes=False),
    )


def _stage_c_body(part_ref, nh_ref, wr_ref, br_ref, out_ref):
    ps = jnp.sum(part_ref[...], axis=0)
    refine = jnp.sum(nh_ref[...] * wr_ref[...], axis=-1)
    out_ref[...] = (ps.reshape(NB_C) + refine + br_ref[0, 0])[None, :]


def _stage_c(partials, nh2, wr, br):
    grid = (pl.cdiv(N, NB_C),)
    return pl.pallas_call(
        _stage_c_body,
        grid=grid,
        in_specs=[
            pl.BlockSpec((NW, MB_C, 128), lambda i: (0, i, 0)),
            pl.BlockSpec((NB_C, DN), lambda i: (i, 0)),
            pl.BlockSpec((1, DN), lambda i: (0, 0)),
            pl.BlockSpec(memory_space=pltpu.SMEM),
        ],
        out_specs=pl.BlockSpec((1, NB_C), lambda i: (0, i)),
        out_shape=jax.ShapeDtypeStruct((1, N), jnp.float32),
    )(partials, nh2, wr, br)


def kernel(X, node_h, edge_h, edge_idx, mask_i, mask_ij, Wu_w, Wu_b, Wr_w, Wr_b):
    eh2 = edge_h.reshape(N, K * DE)
    mask2 = mask_ij.reshape(N, K)
    x2 = X.reshape(N, K)
    idx2 = edge_idx.reshape(N, K).astype(jnp.int32)
    nh2 = node_h.reshape(N, DN)
    wmat = jnp.kron(jnp.eye(K, dtype=jnp.float32), Wu_w.reshape(DE, 1))
    bu = Wu_b.reshape(1, 1)
    br = Wr_b.reshape(1, 1)

    wT, xT, idxT = _stage_a(eh2, mask2, x2, idx2, wmat, bu)
    partials = _sc_gather()(wT, xT, idxT)
    out2 = _stage_c(partials, nh2, Wr_w.reshape(1, DN), br)
    return out2.reshape(1, N, 1)
